# baseline (speedup 1.0000x reference)
import functools

import jax
import jax.numpy as jnp
from jax import lax
from jax.experimental import pallas as pl
from jax.experimental.pallas import tpu as pltpu
from jax.experimental.pallas import tpu_sc as plsc

_DIM = 64
_LANES = 16
_NCORES = 2
_NSUB = 16
_NW = _NCORES * _NSUB
_NSLOT = 4


def _make_sc_call(batch):
    bpw = batch // _NW
    nblk = bpw // _LANES
    mesh = plsc.VectorSubcoreMesh(core_axis_name="c", subcore_axis_name="s")

    slot_shape = (_LANES // 8, 8, _DIM)

    @functools.partial(
        pl.kernel,
        mesh=mesh,
        out_type=jax.ShapeDtypeStruct((_NW, _LANES), jnp.float32),
        scratch_types=[
            pltpu.VMEM((bpw,), jnp.int32),
            pltpu.VMEM((bpw, _DIM), jnp.float32),
            pltpu.VMEM(slot_shape, jnp.float32),
            pltpu.VMEM(slot_shape, jnp.float32),
            pltpu.VMEM(slot_shape, jnp.float32),
            pltpu.VMEM(slot_shape, jnp.float32),
            pltpu.VMEM((_LANES,), jnp.float32),
            pltpu.SemaphoreType.DMA,
            pltpu.SemaphoreType.DMA,
            pltpu.SemaphoreType.DMA,
            pltpu.SemaphoreType.DMA,
            pltpu.SemaphoreType.DMA,
        ],
    )
    def sc_kernel(x_hbm, y_hbm, centers_hbm, out_hbm, idx_v, x_v,
                  c_v0, c_v1, c_v2, c_v3, acc_v,
                  sem_x, sem_g0, sem_g1, sem_g2, sem_g3):
        wid = lax.axis_index("s") * _NCORES + lax.axis_index("c")
        base = wid * bpw
        slots = ((c_v0, sem_g0), (c_v1, sem_g1),
                 (c_v2, sem_g2), (c_v3, sem_g3))

        pltpu.sync_copy(y_hbm.at[pl.ds(base, bpw)], idx_v)
        cp_x = pltpu.async_copy(x_hbm.at[pl.ds(base, bpw)], x_v, sem_x)

        def issue_block(g, cref, sem):
            rv = idx_v[pl.ds(g * _LANES, _LANES)]
            tv = rv >> 3
            sv = rv & 7
            for i in range(_LANES):
                pltpu.async_copy(
                    centers_hbm.at[tv[i], sv[i]],
                    cref.at[i // 8, i % 8],
                    sem,
                )

        def drain_block(cref, sem):
            pltpu.make_async_copy(
                centers_hbm.at[pl.ds(0, _LANES // 8)], cref, sem).wait()

        def compute_block(g, cref, accs_in):
            off = g * _LANES
            new = list(accs_in)
            for i in range(_LANES):
                for k in range(_DIM // _LANES):
                    d = (x_v[off + i, pl.ds(k * _LANES, _LANES)]
                         - cref[i // 8, i % 8, pl.ds(k * _LANES, _LANES)])
                    new[k] = new[k] + d * d
            return tuple(new)

        for b, (cref, sem) in enumerate(slots):
            issue_block(b, cref, sem)
        cp_x.wait()

        zeros = jnp.zeros((_LANES,), jnp.float32)

        def body(it, accs_in):
            g = it * _NSLOT
            accs = accs_in
            for b, (cref, sem) in enumerate(slots):
                drain_block(cref, sem)
                accs = compute_block(g + b, cref, accs)

                @pl.when(g + b + _NSLOT < nblk)
                def _():
                    issue_block(g + b + _NSLOT, cref, sem)

            return accs

        accs = lax.fori_loop(0, nblk // _NSLOT, body,
                             (zeros, zeros, zeros, zeros))

        acc_v[...] = accs[0] + accs[1] + accs[2] + accs[3]
        pltpu.sync_copy(acc_v, out_hbm.at[wid])

    return sc_kernel


def kernel(x, y, centers):
    batch, dim = x.shape
    nrows = centers.shape[0]
    centers3 = centers.reshape(nrows // 8, 8, dim)
    partials = _make_sc_call(batch)(x, y.astype(jnp.int32), centers3)
    return jnp.sum(partials) / (batch * dim)
